# Initial kernel scaffold; baseline (speedup 1.0000x reference)
import jax
import jax.numpy as jnp
from jax import lax
from jax.experimental import pallas as pl
from jax.experimental.pallas import tpu as pltpu
from jax.experimental.pallas import tpu_sc as plsc
import functools

NC = 2
NS = 16
L = 16

N = 10000
B = 8
NP = 1250
D = 128

E_REAL = 320000
E_AUG = E_REAL + N
ROWS = 2592
CH = ROWS // (NC * NS)
E_PAD = ROWS * 128

HP = 10240
HSL = HP // NS
GP = 80640
GSL = GP // NS

MAGIC = 6711
SHIFT = 23

_mesh = plsc.VectorSubcoreMesh(
    core_axis_name="c", subcore_axis_name="s", num_cores=NC, num_subcores=NS)


def _zero_vmem(ref, n):
    z = jnp.zeros((L,), jnp.float32)

    def body(i, _):
        ref[pl.ds(i * L, L)] = z
        return 0

    lax.fori_loop(0, n // L, body, 0)


def _fast_rsqrt(d):
    ii = lax.bitcast_convert_type(d, jnp.int32)
    ii = jnp.int32(0x5F3759DF) - (ii >> 1)
    y = lax.bitcast_convert_type(ii, jnp.float32)
    for _ in range(3):
        y = y * (1.5 - 0.5 * d * y * y)
    return y


@functools.partial(
    pl.kernel,
    out_type=jax.ShapeDtypeStruct((NC * HP,), jnp.float32),
    mesh=_mesh,
    scratch_types=[
        pltpu.VMEM((CH, 128), jnp.int32),
        pltpu.VMEM((128,), jnp.float32),
        pltpu.VMEM((HSL,), jnp.float32),
        pltpu.VMEM_SHARED((HP,), jnp.float32),
    ],
)
def _k1_hist(dst_hbm, hist_out, dstv, ones, zbuf, hist_sh):
    c = lax.axis_index("c")
    s = lax.axis_index("s")
    tid = c * NS + s

    for k in range(128 // L):
        ones[pl.ds(k * L, L)] = jnp.ones((L,), jnp.float32)
    _zero_vmem(zbuf, HSL)
    off = pl.multiple_of(s * HSL, 8)
    pltpu.sync_copy(zbuf, hist_sh.at[pl.ds(off, HSL)])
    pltpu.sync_copy(dst_hbm.at[pl.ds(tid * CH, CH)], dstv)
    plsc.subcore_barrier()

    def body(j, _):
        pltpu.sync_copy(ones, hist_sh.at[dstv.at[j]], add=True)
        return 0

    lax.fori_loop(0, CH, body, 0)
    plsc.subcore_barrier()
    hoff = pl.multiple_of(c * HP + s * HSL, 8)
    pltpu.sync_copy(hist_sh.at[pl.ds(off, HSL)], hist_out.at[pl.ds(hoff, HSL)])


@functools.partial(
    pl.kernel,
    out_type=jax.ShapeDtypeStruct((NC * GP,), jnp.float32),
    mesh=_mesh,
    scratch_types=[
        pltpu.VMEM((CH, 128), jnp.int32),
        pltpu.VMEM((CH, 128), jnp.int32),
        pltpu.VMEM((NC * HP,), jnp.float32),
        pltpu.VMEM((HP,), jnp.float32),
        pltpu.VMEM((HP,), jnp.float32),
        pltpu.VMEM((GSL,), jnp.float32),
        pltpu.VMEM((128,), jnp.float32),
        pltpu.VMEM((128,), jnp.int32),
        pltpu.VMEM_SHARED((GP,), jnp.float32),
    ],
)
def _k2_scatter(src_hbm, dst_hbm, hist_hbm, v_hbm, g_out,
                srcv, dstv, histv, vv, dinvv, zbuf, wrow, irow, g_sh):
    c = lax.axis_index("c")
    s = lax.axis_index("s")
    tid = c * NS + s

    _zero_vmem(zbuf, GSL)
    goff = pl.multiple_of(s * GSL, 8)
    pltpu.sync_copy(zbuf, g_sh.at[pl.ds(goff, GSL)])
    pltpu.sync_copy(hist_hbm, histv)
    pltpu.sync_copy(v_hbm, vv)
    pltpu.sync_copy(src_hbm.at[pl.ds(tid * CH, CH)], srcv)
    pltpu.sync_copy(dst_hbm.at[pl.ds(tid * CH, CH)], dstv)

    def dbody(i, _):
        sl = pl.ds(i * L, L)
        deg = histv[pl.ds(i * L, L)] + histv[pl.ds(HP + i * L, L)]
        dinvv[sl] = _fast_rsqrt(deg)
        return 0

    lax.fori_loop(0, HP // L, dbody, 0)
    plsc.subcore_barrier()

    def ebody(j, _):
        for k in range(128 // L):
            sl = pl.ds(k * L, L)
            s16 = srcv[j, sl]
            d16 = dstv[j, sl]
            dis = plsc.load_gather(dinvv, [s16])
            did = plsc.load_gather(dinvv, [d16])
            vd = plsc.load_gather(vv, [d16])
            wrow[sl] = vd * dis * did
            irow[sl] = (d16 * MAGIC >> SHIFT) * N + s16
        pltpu.sync_copy(wrow, g_sh.at[irow], add=True)
        return 0

    lax.fori_loop(0, CH, ebody, 0)
    plsc.subcore_barrier()
    ooff = pl.multiple_of(c * GP + s * GSL, 8)
    pltpu.sync_copy(g_sh.at[pl.ds(goff, GSL)], g_out.at[pl.ds(ooff, GSL)])


def _k3_body(g_ref, x_ref, pm_ref, w_ref, b_ref, o_ref):
    g = g_ref[0] + g_ref[1]
    gx = jnp.dot(g, x_ref[...], preferred_element_type=jnp.float32)
    r = jnp.dot(gx, w_ref[...], preferred_element_type=jnp.float32)
    msum = jnp.sum(pm_ref[...], axis=1)
    o_ref[...] = r + msum[:, None] * b_ref[...][None, :]


_k3_tail = pl.pallas_call(
    _k3_body,
    out_shape=jax.ShapeDtypeStruct((B, D), jnp.float32),
)


def kernel(x, edge_index, protein_mask, W, b):
    src = edge_index[0].astype(jnp.int32)
    dst = edge_index[1].astype(jnp.int32)
    loop = jnp.arange(N, dtype=jnp.int32)
    npad = E_PAD - E_AUG
    src_a = jnp.concatenate([src, loop, jnp.zeros((npad,), jnp.int32)])
    dst_a = jnp.concatenate([dst, loop, jnp.full((npad,), N, jnp.int32)])
    src2 = src_a.reshape(ROWS, 128)
    dst2 = dst_a.reshape(ROWS, 128)
    v = jnp.concatenate(
        [protein_mask.reshape(-1), jnp.zeros((HP - N,), jnp.float32)])

    hist = _k1_hist(dst2)
    gflat = _k2_scatter(src2, dst2, hist, v)
    g3 = gflat.reshape(NC, GP)[:, :B * N].reshape(NC, B, N)
    return _k3_tail(g3, x, protein_mask, W, b)


# speedup vs baseline: 76.6726x; 76.6726x over previous
import jax
import jax.numpy as jnp
from jax import lax
from jax.experimental import pallas as pl
from jax.experimental.pallas import tpu as pltpu
from jax.experimental.pallas import tpu_sc as plsc
import functools

NC = 2
NS = 16
L = 16

N = 10000
B = 8
NP = 1250
D = 128

E_REAL = 320000
E_AUG = E_REAL + N
CH = 88
ROWS = CH * NC * NS
E_PAD = ROWS * 128
RR = -(-E_AUG // 128)

HP = 10240
HSL = HP // NS
GP = 81920
GSL = GP // NS

MAGIC = 6711
SHIFT = 23

_mesh = plsc.VectorSubcoreMesh(
    core_axis_name="c", subcore_axis_name="s", num_cores=NC, num_subcores=NS)
_sc_params = pltpu.CompilerParams(needs_layout_passes=False)


def _zero_vmem(ref, n):
    z = jnp.zeros((L,), jnp.float32)

    def body(i, _):
        ref[pl.ds(i * L, L)] = z
        return 0

    lax.fori_loop(0, n // L, body, 0)


def _fast_rsqrt(d):
    ii = lax.bitcast_convert_type(d, jnp.int32)
    ii = jnp.int32(0x5F3759DF) - (ii >> 1)
    y = lax.bitcast_convert_type(ii, jnp.float32)
    for _ in range(3):
        y = y * (1.5 - 0.5 * d * y * y)
    return y


@functools.partial(
    pl.kernel,
    out_type=jax.ShapeDtypeStruct((NC * HP,), jnp.float32),
    mesh=_mesh,
    compiler_params=_sc_params,
    scratch_types=[
        pltpu.VMEM((CH, 128), jnp.int32),
        pltpu.VMEM((128,), jnp.float32),
        pltpu.VMEM((HSL,), jnp.float32),
        pltpu.VMEM_SHARED((HP,), jnp.float32),
    ],
)
def _k1_hist(dst_hbm, hist_out, dstv, ones, zbuf, hist_sh):
    c = lax.axis_index("c")
    s = lax.axis_index("s")
    tid = c * NS + s

    for k in range(128 // L):
        ones[pl.ds(k * L, L)] = jnp.ones((L,), jnp.float32)
    _zero_vmem(zbuf, HSL)
    off = pl.multiple_of(s * HSL, 16)
    pltpu.sync_copy(zbuf, hist_sh.at[pl.ds(off, HSL)])
    pltpu.sync_copy(dst_hbm.at[pl.ds(tid * CH, CH)], dstv)
    plsc.subcore_barrier()
    nreal = jnp.clip(RR - tid * CH, 0, CH)

    def body(j, _):
        pltpu.sync_copy(ones, hist_sh.at[dstv.at[j]], add=True)
        return 0

    lax.fori_loop(0, nreal, body, 0)
    plsc.subcore_barrier()
    hoff = pl.multiple_of(c * HP + s * HSL, 16)
    pltpu.sync_copy(hist_sh.at[pl.ds(off, HSL)], hist_out.at[pl.ds(hoff, HSL)])


@functools.partial(
    pl.kernel,
    out_type=jax.ShapeDtypeStruct((NC * GP,), jnp.float32),
    mesh=_mesh,
    compiler_params=_sc_params,
    scratch_types=[
        pltpu.VMEM((CH, 128), jnp.int32),
        pltpu.VMEM((CH, 128), jnp.int32),
        pltpu.VMEM((NC * HP,), jnp.float32),
        pltpu.VMEM((HP,), jnp.float32),
        pltpu.VMEM((HP,), jnp.float32),
        pltpu.VMEM((GSL,), jnp.float32),
        pltpu.VMEM((128,), jnp.float32),
        pltpu.VMEM((128,), jnp.int32),
        pltpu.VMEM_SHARED((GP,), jnp.float32),
    ],
)
def _k2_scatter(src_hbm, dst_hbm, hist_hbm, v_hbm, g_out,
                srcv, dstv, histv, vv, dinvv, zbuf, wrow, irow, g_sh):
    c = lax.axis_index("c")
    s = lax.axis_index("s")
    tid = c * NS + s

    _zero_vmem(zbuf, GSL)
    goff = pl.multiple_of(s * GSL, 128)
    pltpu.sync_copy(zbuf, g_sh.at[pl.ds(goff, GSL)])
    pltpu.sync_copy(hist_hbm, histv)
    pltpu.sync_copy(v_hbm, vv)
    pltpu.sync_copy(src_hbm.at[pl.ds(tid * CH, CH)], srcv)
    pltpu.sync_copy(dst_hbm.at[pl.ds(tid * CH, CH)], dstv)

    def dbody(i, _):
        sl = pl.ds(i * L, L)
        deg = histv[pl.ds(i * L, L)] + histv[pl.ds(HP + i * L, L)]
        dinvv[sl] = _fast_rsqrt(deg)
        return 0

    lax.fori_loop(0, HP // L, dbody, 0)
    plsc.subcore_barrier()
    nreal = jnp.clip(RR - tid * CH, 0, CH)

    def ebody(j, _):
        for k in range(128 // L):
            sl = pl.ds(k * L, L)
            s16 = srcv[j, sl]
            d16 = dstv[j, sl]
            dis = plsc.load_gather(dinvv, [s16])
            did = plsc.load_gather(dinvv, [d16])
            vd = plsc.load_gather(vv, [d16])
            wrow[sl] = vd * dis * did
            irow[sl] = (d16 * MAGIC >> SHIFT) * N + s16
        pltpu.sync_copy(wrow, g_sh.at[irow], add=True)
        return 0

    lax.fori_loop(0, nreal, ebody, 0)
    plsc.subcore_barrier()
    ooff = pl.multiple_of(c * GP + s * GSL, 128)
    pltpu.sync_copy(g_sh.at[pl.ds(goff, GSL)], g_out.at[pl.ds(ooff, GSL)])


def _k3_body(g_ref, x_ref, pm_ref, w_ref, b_ref, o_ref):
    g = g_ref[0] + g_ref[1]
    gx = jnp.dot(g, x_ref[...], preferred_element_type=jnp.float32)
    r = jnp.dot(gx, w_ref[...], preferred_element_type=jnp.float32)
    msum = jnp.sum(pm_ref[...], axis=1)
    o_ref[...] = r + msum[:, None] * b_ref[...][None, :]


_k3_tail = pl.pallas_call(
    _k3_body,
    out_shape=jax.ShapeDtypeStruct((B, D), jnp.float32),
)


def kernel(x, edge_index, protein_mask, W, b):
    src = edge_index[0].astype(jnp.int32)
    dst = edge_index[1].astype(jnp.int32)
    loop = jnp.arange(N, dtype=jnp.int32)
    npad = E_PAD - E_AUG
    src_a = jnp.concatenate([src, loop, jnp.zeros((npad,), jnp.int32)])
    dst_a = jnp.concatenate([dst, loop, jnp.full((npad,), N, jnp.int32)])
    src2 = src_a.reshape(ROWS, 128)
    dst2 = dst_a.reshape(ROWS, 128)
    v = jnp.concatenate(
        [protein_mask.reshape(-1), jnp.zeros((HP - N,), jnp.float32)])

    hist = _k1_hist(dst2)
    gflat = _k2_scatter(src2, dst2, hist, v)
    g3 = gflat.reshape(NC, GP)[:, :B * N].reshape(NC, B, N)
    return _k3_tail(g3, x, protein_mask, W, b)


# speedup vs baseline: 105.8916x; 1.3811x over previous
import jax
import jax.numpy as jnp
from jax import lax
from jax.experimental import pallas as pl
from jax.experimental.pallas import tpu as pltpu
from jax.experimental.pallas import tpu_sc as plsc
import functools

NC = 2
NS = 16
L = 16

N = 10000
B = 8
NP = 1250
D = 128

E_REAL = 320000
E_AUG = E_REAL + N
CH = 88
NB = 8
ROWS = CH * NC * NS
E_PAD = ROWS * 128

HP = 10240
HSL = HP // NS
GP = 81920
GSL = GP // NS

MAGIC = 6711
SHIFT = 23

_mesh = plsc.VectorSubcoreMesh(
    core_axis_name="c", subcore_axis_name="s", num_cores=NC, num_subcores=NS)
_sc_params = pltpu.CompilerParams(needs_layout_passes=False)


def _zero_vmem(ref, n):
    z = jnp.zeros((L,), jnp.float32)

    def body(i, _):
        ref[pl.ds(i * L, L)] = z
        return 0

    lax.fori_loop(0, n // L, body, 0)


def _fast_rsqrt(d):
    ii = lax.bitcast_convert_type(d, jnp.int32)
    ii = jnp.int32(0x5F3759DF) - (ii >> 1)
    y = lax.bitcast_convert_type(ii, jnp.float32)
    for _ in range(3):
        y = y * (1.5 - 0.5 * d * y * y)
    return y


@functools.partial(
    pl.kernel,
    out_type=jax.ShapeDtypeStruct((NC * HP,), jnp.float32),
    mesh=_mesh,
    compiler_params=_sc_params,
    scratch_types=[
        pltpu.VMEM((CH, 128), jnp.int32),
        pltpu.VMEM((128,), jnp.float32),
        pltpu.VMEM((HSL,), jnp.float32),
        pltpu.VMEM_SHARED((HP,), jnp.float32),
        pltpu.SemaphoreType.DMA,
    ],
)
def _k1_hist(dst_hbm, hist_out, dstv, ones, zbuf, hist_sh, sem):
    c = lax.axis_index("c")
    s = lax.axis_index("s")
    tid = c * NS + s

    for k in range(128 // L):
        ones[pl.ds(k * L, L)] = jnp.ones((L,), jnp.float32)
    _zero_vmem(zbuf, HSL)
    off = pl.multiple_of(s * HSL, 128)
    pltpu.sync_copy(zbuf, hist_sh.at[pl.ds(off, HSL)])
    pltpu.sync_copy(dst_hbm.at[pl.ds(tid * CH, CH)], dstv)
    plsc.subcore_barrier()

    def blk(jb, _):
        descs = []
        for t in range(NB):
            descs.append(pltpu.async_copy(
                ones, hist_sh.at[dstv.at[jb * NB + t]], sem, add=True))
        for dsc in descs:
            dsc.wait()
        return 0

    lax.fori_loop(0, CH // NB, blk, 0)
    plsc.subcore_barrier()
    hoff = pl.multiple_of(c * HP + s * HSL, 128)
    pltpu.sync_copy(hist_sh.at[pl.ds(off, HSL)], hist_out.at[pl.ds(hoff, HSL)])


@functools.partial(
    pl.kernel,
    out_type=jax.ShapeDtypeStruct((NC * GP,), jnp.float32),
    mesh=_mesh,
    compiler_params=_sc_params,
    scratch_types=[
        pltpu.VMEM((CH, 128), jnp.int32),
        pltpu.VMEM((CH, 128), jnp.int32),
        pltpu.VMEM((HSL,), jnp.float32),
        pltpu.VMEM((HSL,), jnp.float32),
        pltpu.VMEM((HSL,), jnp.float32),
        pltpu.VMEM((HSL,), jnp.float32),
        pltpu.VMEM((HSL,), jnp.float32),
        pltpu.VMEM((HP,), jnp.float32),
        pltpu.VMEM((HP,), jnp.float32),
        pltpu.VMEM((GSL,), jnp.float32),
        pltpu.VMEM((NB, 128), jnp.float32),
        pltpu.VMEM((NB, 128), jnp.int32),
        pltpu.VMEM_SHARED((HP,), jnp.float32),
        pltpu.VMEM_SHARED((HP,), jnp.float32),
        pltpu.VMEM_SHARED((GP,), jnp.float32),
        pltpu.SemaphoreType.DMA,
    ],
)
def _k2_scatter(src_hbm, dst_hbm, hist_hbm, v_hbm, g_out,
                srcv, dstv, h0v, h1v, vslv, dslv, aslv, dinvv, av, zbuf,
                wrows, irows, dinv_sh, a_sh, g_sh, sem):
    c = lax.axis_index("c")
    s = lax.axis_index("s")
    tid = c * NS + s

    _zero_vmem(zbuf, GSL)
    goff = pl.multiple_of(s * GSL, 128)
    pltpu.sync_copy(zbuf, g_sh.at[pl.ds(goff, GSL)])

    hoff = pl.multiple_of(s * HSL, 128)
    pltpu.sync_copy(hist_hbm.at[pl.ds(hoff, HSL)], h0v)
    hoff1 = pl.multiple_of(HP + s * HSL, 128)
    pltpu.sync_copy(hist_hbm.at[pl.ds(hoff1, HSL)], h1v)
    pltpu.sync_copy(v_hbm.at[pl.ds(hoff, HSL)], vslv)

    def dbody(i, _):
        sl = pl.ds(i * L, L)
        y = _fast_rsqrt(h0v[sl] + h1v[sl])
        dslv[sl] = y
        aslv[sl] = vslv[sl] * y
        return 0

    lax.fori_loop(0, HSL // L, dbody, 0)
    pltpu.sync_copy(dslv, dinv_sh.at[pl.ds(hoff, HSL)])
    pltpu.sync_copy(aslv, a_sh.at[pl.ds(hoff, HSL)])

    pltpu.sync_copy(src_hbm.at[pl.ds(tid * CH, CH)], srcv)
    pltpu.sync_copy(dst_hbm.at[pl.ds(tid * CH, CH)], dstv)

    plsc.subcore_barrier()
    pltpu.sync_copy(dinv_sh, dinvv)
    pltpu.sync_copy(a_sh, av)

    def blk(jb, _):
        descs = []
        for t in range(NB):
            row = jb * NB + t
            for k in range(128 // L):
                sl = pl.ds(k * L, L)
                s16 = srcv[row, sl]
                d16 = dstv[row, sl]
                dis = plsc.load_gather(dinvv, [s16])
                ad = plsc.load_gather(av, [d16])
                wrows[t, sl] = ad * dis
                irows[t, sl] = (d16 * MAGIC >> SHIFT) * N + s16
            descs.append(pltpu.async_copy(
                wrows.at[t], g_sh.at[irows.at[t]], sem, add=True))
        for dsc in descs:
            dsc.wait()
        return 0

    lax.fori_loop(0, CH // NB, blk, 0)
    plsc.subcore_barrier()
    ooff = pl.multiple_of(c * GP + s * GSL, 128)
    pltpu.sync_copy(g_sh.at[pl.ds(goff, GSL)], g_out.at[pl.ds(ooff, GSL)])


def _k3_body(g_ref, x_ref, pm_ref, w_ref, b_ref, o_ref):
    g = g_ref[0] + g_ref[1]
    gx = jnp.dot(g, x_ref[...], preferred_element_type=jnp.float32)
    r = jnp.dot(gx, w_ref[...], preferred_element_type=jnp.float32)
    msum = jnp.sum(pm_ref[...], axis=1)
    o_ref[...] = r + msum[:, None] * b_ref[...][None, :]


_k3_tail = pl.pallas_call(
    _k3_body,
    out_shape=jax.ShapeDtypeStruct((B, D), jnp.float32),
)


def kernel(x, edge_index, protein_mask, W, b):
    src = edge_index[0].astype(jnp.int32)
    dst = edge_index[1].astype(jnp.int32)
    loop = jnp.arange(N, dtype=jnp.int32)
    npad = E_PAD - E_AUG
    j = jnp.arange(npad, dtype=jnp.int32)
    src_a = jnp.concatenate([src, loop, j % 1920])
    dst_a = jnp.concatenate([dst, loop, N + (j % 240)])
    src2 = src_a.reshape(ROWS, 128)
    dst2 = dst_a.reshape(ROWS, 128)
    v = jnp.concatenate(
        [protein_mask.reshape(-1), jnp.zeros((HP - N,), jnp.float32)])

    hist = _k1_hist(dst2)
    gflat = _k2_scatter(src2, dst2, hist, v)
    g3 = gflat.reshape(NC, GP)[:, :B * N].reshape(NC, B, N)
    return _k3_tail(g3, x, protein_mask, W, b)
